# initial kernel scaffold (unmeasured)
import jax
import jax.numpy as jnp
from jax import lax
from jax.experimental import pallas as pl
from jax.experimental.pallas import tpu as pltpu

N_DEV = 32
LOG = 5
ROWS = 1024
COLS = 1024
CHUNK = ROWS // N_DEV


def kernel(x, w_mat):
    def body(x_ref, w_ref, out_ref, r0, r1, r2, r3, r4, send_sems, recv_sems):
        my = lax.axis_index("i")
        recv_refs = [r0, r1, r2, r3, r4]

        out_ref[...] = jnp.dot(
            x_ref[...], w_ref[...], preferred_element_type=jnp.float32
        )

        off = jnp.int32(0)
        for s in range(LOG):
            half = ROWS >> (s + 1)
            m = 1 << (LOG - 1 - s)
            bit = (my >> (LOG - 1 - s)) & 1
            partner = my ^ m
            send_off = off + (1 - bit) * half
            keep_off = off + bit * half
            rdma = pltpu.make_async_remote_copy(
                src_ref=out_ref.at[pl.ds(send_off, half)],
                dst_ref=recv_refs[s],
                send_sem=send_sems.at[s],
                recv_sem=recv_sems.at[s],
                device_id=(partner,),
                device_id_type=pl.DeviceIdType.MESH,
            )
            rdma.start()
            rdma.wait()
            out_ref[pl.ds(keep_off, half), :] = (
                out_ref[pl.ds(keep_off, half), :] + recv_refs[s][...]
            )
            off = keep_off

        for t in range(LOG):
            ln = CHUNK << t
            bit = (my >> t) & 1
            partner = my ^ (1 << t)
            rdma = pltpu.make_async_remote_copy(
                src_ref=out_ref.at[pl.ds(off, ln)],
                dst_ref=out_ref.at[pl.ds(off, ln)],
                send_sem=send_sems.at[LOG + t],
                recv_sem=recv_sems.at[LOG + t],
                device_id=(partner,),
                device_id_type=pl.DeviceIdType.MESH,
            )
            rdma.start()
            rdma.wait()
            off = off - bit * ln

    return pl.pallas_call(
        body,
        out_shape=jax.ShapeDtypeStruct((ROWS, COLS), jnp.float32),
        in_specs=[
            pl.BlockSpec(memory_space=pltpu.VMEM),
            pl.BlockSpec(memory_space=pltpu.VMEM),
        ],
        out_specs=pl.BlockSpec(memory_space=pltpu.VMEM),
        scratch_shapes=[
            pltpu.VMEM((ROWS >> 1, COLS), jnp.float32),
            pltpu.VMEM((ROWS >> 2, COLS), jnp.float32),
            pltpu.VMEM((ROWS >> 3, COLS), jnp.float32),
            pltpu.VMEM((ROWS >> 4, COLS), jnp.float32),
            pltpu.VMEM((ROWS >> 5, COLS), jnp.float32),
            pltpu.SemaphoreType.DMA((2 * LOG,)),
            pltpu.SemaphoreType.DMA((2 * LOG,)),
        ],
        compiler_params=pltpu.CompilerParams(collective_id=0),
    )(x, w_mat)


# baseline (device time: 176972 ns/iter reference)
import jax
import jax.numpy as jnp
from jax import lax
from jax.experimental import pallas as pl
from jax.experimental.pallas import tpu as pltpu

N_DEV = 32
LOG = 5
ROWS = 1024
COLS = 1024
CHUNK = ROWS // N_DEV


def kernel(x, w_mat):
    def body(x_ref, w_ref, out_ref, r0, r1, r2, r3, r4, send_sems, recv_sems):
        my = lax.axis_index("i")
        recv_refs = [r0, r1, r2, r3, r4]

        out_ref[...] = jnp.dot(
            x_ref[...], w_ref[...], preferred_element_type=jnp.float32
        )

        off = jnp.int32(0)
        for s in range(LOG):
            half = ROWS >> (s + 1)
            m = 1 << (LOG - 1 - s)
            bit = (my >> (LOG - 1 - s)) & 1
            partner = my ^ m
            send_off = off + (1 - bit) * half
            keep_off = off + bit * half
            rdma = pltpu.make_async_remote_copy(
                src_ref=out_ref.at[pl.ds(send_off, half)],
                dst_ref=recv_refs[s],
                send_sem=send_sems.at[s],
                recv_sem=recv_sems.at[s],
                device_id=(partner,),
                device_id_type=pl.DeviceIdType.MESH,
            )
            rdma.start()
            rdma.wait()
            out_ref[pl.ds(keep_off, half), :] = (
                out_ref[pl.ds(keep_off, half), :] + recv_refs[s][...]
            )
            off = keep_off

        for t in range(LOG):
            ln = CHUNK << t
            bit = (my >> t) & 1
            partner = my ^ (1 << t)
            rdma = pltpu.make_async_remote_copy(
                src_ref=out_ref.at[pl.ds(off, ln)],
                dst_ref=out_ref.at[pl.ds(off, ln)],
                send_sem=send_sems.at[LOG + t],
                recv_sem=recv_sems.at[LOG + t],
                device_id=(partner,),
                device_id_type=pl.DeviceIdType.MESH,
            )
            rdma.start()
            rdma.wait()
            off = off - bit * ln

    return pl.pallas_call(
        body,
        out_shape=jax.ShapeDtypeStruct((ROWS, COLS), jnp.float32),
        in_specs=[
            pl.BlockSpec(memory_space=pltpu.VMEM),
            pl.BlockSpec(memory_space=pltpu.VMEM),
        ],
        out_specs=pl.BlockSpec(memory_space=pltpu.VMEM),
        scratch_shapes=[
            pltpu.VMEM((ROWS >> 1, COLS), jnp.float32),
            pltpu.VMEM((ROWS >> 2, COLS), jnp.float32),
            pltpu.VMEM((ROWS >> 3, COLS), jnp.float32),
            pltpu.VMEM((ROWS >> 4, COLS), jnp.float32),
            pltpu.VMEM((ROWS >> 5, COLS), jnp.float32),
            pltpu.SemaphoreType.DMA((2 * LOG,)),
            pltpu.SemaphoreType.DMA((2 * LOG,)),
        ],
    )(x, w_mat)


# device time: 79649 ns/iter; 2.2219x vs baseline; 2.2219x over previous
import jax
import jax.numpy as jnp
from jax import lax
from jax.experimental import pallas as pl
from jax.experimental.pallas import tpu as pltpu

N_DEV = 32
LOG = 5
ROWS = 1024
COLS = 1024
CHUNK = ROWS // N_DEV


def kernel(x, w_mat):
    def body(
        x_ref, w_ref, out_ref,
        rsend, r0, r1, r2, r3, r4, ag,
        send_sems, recv_sems,
    ):
        my = lax.axis_index("i")
        z = my >> 3
        idx = my & 7
        yc = idx >> 1
        xc = (idx & 1) ^ (yc & 1)

        recv_refs = [r0, r1, r2, r3, r4]

        out_ref[...] = jnp.dot(
            x_ref[...], w_ref[...], preferred_element_type=jnp.float32
        )

        py = yc ^ 1
        y1_partner = 8 * z + 2 * py + (xc ^ (py & 1))

        stages = [
            (my ^ 1, xc),
            (my ^ 8, z & 1),
            (y1_partner, yc & 1),
            (my ^ 16, (z >> 1) & 1),
            (my ^ 4, (yc >> 1) & 1),
        ]

        off = jnp.int32(0)
        for s, (partner, bit) in enumerate(stages):
            half = ROWS >> (s + 1)
            send_off = off + (1 - bit) * half
            keep_off = off + bit * half
            rsend[pl.ds(send_off, half), :] = out_ref[
                pl.ds(send_off, half), :
            ].astype(jnp.bfloat16)
            rdma = pltpu.make_async_remote_copy(
                src_ref=rsend.at[pl.ds(send_off, half)],
                dst_ref=recv_refs[s],
                send_sem=send_sems.at[s],
                recv_sem=recv_sems.at[s],
                device_id=(partner,),
                device_id_type=pl.DeviceIdType.MESH,
            )
            rdma.start()
            rdma.wait()
            out_ref[pl.ds(keep_off, half), :] = (
                out_ref[pl.ds(keep_off, half), :]
                + recv_refs[s][...].astype(jnp.float32)
            )
            off = keep_off

        ag[pl.ds(off, CHUNK), :] = out_ref[pl.ds(off, CHUNK), :].astype(
            jnp.bfloat16
        )
        for t, (partner, bit) in enumerate(reversed(stages)):
            ln = CHUNK << t
            rdma = pltpu.make_async_remote_copy(
                src_ref=ag.at[pl.ds(off, ln)],
                dst_ref=ag.at[pl.ds(off, ln)],
                send_sem=send_sems.at[LOG + t],
                recv_sem=recv_sems.at[LOG + t],
                device_id=(partner,),
                device_id_type=pl.DeviceIdType.MESH,
            )
            rdma.start()
            rdma.wait()
            off = off - bit * ln

        out_ref[...] = ag[...].astype(jnp.float32)

    return pl.pallas_call(
        body,
        out_shape=jax.ShapeDtypeStruct((ROWS, COLS), jnp.float32),
        in_specs=[
            pl.BlockSpec(memory_space=pltpu.VMEM),
            pl.BlockSpec(memory_space=pltpu.VMEM),
        ],
        out_specs=pl.BlockSpec(memory_space=pltpu.VMEM),
        scratch_shapes=[
            pltpu.VMEM((ROWS, COLS), jnp.bfloat16),
            pltpu.VMEM((ROWS >> 1, COLS), jnp.bfloat16),
            pltpu.VMEM((ROWS >> 2, COLS), jnp.bfloat16),
            pltpu.VMEM((ROWS >> 3, COLS), jnp.bfloat16),
            pltpu.VMEM((ROWS >> 4, COLS), jnp.bfloat16),
            pltpu.VMEM((ROWS >> 5, COLS), jnp.bfloat16),
            pltpu.VMEM((ROWS, COLS), jnp.bfloat16),
            pltpu.SemaphoreType.DMA((2 * LOG,)),
            pltpu.SemaphoreType.DMA((2 * LOG,)),
        ],
    )(x, w_mat)


# device time: 59264 ns/iter; 2.9862x vs baseline; 1.3440x over previous
import jax
import jax.numpy as jnp
from jax import lax
from jax.experimental import pallas as pl
from jax.experimental.pallas import tpu as pltpu

N_DEV = 32
LOG = 5
ROWS = 1024
COLS = 1024
HALF_ROWS = ROWS // 2
CHUNK = HALF_ROWS // N_DEV


def kernel(x, w_mat):
    def body(
        x_ref, w_ref, out_ref,
        rsend,
        ra0, ra1, ra2, ra3, ra4,
        rb0, rb1, rb2, rb3, rb4,
        ag,
        send_sems, recv_sems,
    ):
        my = lax.axis_index("i")
        z = my >> 3
        idx = my & 7
        yc = idx >> 1
        xc = (idx & 1) ^ (yc & 1)

        recv_a = [ra0, ra1, ra2, ra3, ra4]
        recv_b = [rb0, rb1, rb2, rb3, rb4]

        out_ref[...] = jnp.dot(
            x_ref[...].astype(jnp.bfloat16),
            w_ref[...].astype(jnp.bfloat16),
            preferred_element_type=jnp.float32,
        )

        py = yc ^ 1
        y1_partner = 8 * z + 2 * py + (xc ^ (py & 1))

        x_st = (my ^ 1, xc)
        z1_st = (my ^ 8, z & 1)
        y1_st = (y1_partner, yc & 1)
        z2_st = (my ^ 16, (z >> 1) & 1)
        y2_st = (my ^ 4, (yc >> 1) & 1)
        stages_a = [x_st, z1_st, y1_st, z2_st, y2_st]
        stages_b = [y1_st, x_st, z1_st, y2_st, z2_st]

        def rs_stage(s, off, stage, recv_ref, sem_base):
            partner, bit = stage
            half = HALF_ROWS >> (s + 1)
            send_off = off + (1 - bit) * half
            keep_off = off + bit * half
            rsend[pl.ds(send_off, half), :] = out_ref[
                pl.ds(send_off, half), :
            ].astype(jnp.bfloat16)
            rdma = pltpu.make_async_remote_copy(
                src_ref=rsend.at[pl.ds(send_off, half)],
                dst_ref=recv_ref,
                send_sem=send_sems.at[sem_base + s],
                recv_sem=recv_sems.at[sem_base + s],
                device_id=(partner,),
                device_id_type=pl.DeviceIdType.MESH,
            )
            rdma.start()
            return rdma, keep_off, half

        off_a = jnp.int32(0)
        off_b = jnp.int32(HALF_ROWS)
        for s in range(LOG):
            rdma_a, keep_a, half = rs_stage(s, off_a, stages_a[s], recv_a[s], 0)
            rdma_b, keep_b, _ = rs_stage(s, off_b, stages_b[s], recv_b[s], LOG)
            rdma_a.wait()
            out_ref[pl.ds(keep_a, half), :] = (
                out_ref[pl.ds(keep_a, half), :]
                + recv_a[s][...].astype(jnp.float32)
            )
            rdma_b.wait()
            out_ref[pl.ds(keep_b, half), :] = (
                out_ref[pl.ds(keep_b, half), :]
                + recv_b[s][...].astype(jnp.float32)
            )
            off_a = keep_a
            off_b = keep_b

        ag[pl.ds(off_a, CHUNK), :] = out_ref[pl.ds(off_a, CHUNK), :].astype(
            jnp.bfloat16
        )
        ag[pl.ds(off_b, CHUNK), :] = out_ref[pl.ds(off_b, CHUNK), :].astype(
            jnp.bfloat16
        )

        def ag_stage(t, off, stage, sem_base):
            partner, bit = stage
            ln = CHUNK << t
            rdma = pltpu.make_async_remote_copy(
                src_ref=ag.at[pl.ds(off, ln)],
                dst_ref=ag.at[pl.ds(off, ln)],
                send_sem=send_sems.at[2 * LOG + sem_base + t],
                recv_sem=recv_sems.at[2 * LOG + sem_base + t],
                device_id=(partner,),
                device_id_type=pl.DeviceIdType.MESH,
            )
            rdma.start()
            return rdma, bit, ln

        ag_a = list(reversed(stages_a))
        ag_b = list(reversed(stages_b))
        for t in range(LOG):
            rdma_a, bit_a, ln = ag_stage(t, off_a, ag_a[t], 0)
            rdma_b, bit_b, _ = ag_stage(t, off_b, ag_b[t], LOG)
            rdma_a.wait()
            rdma_b.wait()
            off_a = off_a - bit_a * ln
            off_b = off_b - bit_b * ln

        out_ref[...] = ag[...].astype(jnp.float32)

    rows_of = lambda s: HALF_ROWS >> (s + 1)
    return pl.pallas_call(
        body,
        out_shape=jax.ShapeDtypeStruct((ROWS, COLS), jnp.float32),
        in_specs=[
            pl.BlockSpec(memory_space=pltpu.VMEM),
            pl.BlockSpec(memory_space=pltpu.VMEM),
        ],
        out_specs=pl.BlockSpec(memory_space=pltpu.VMEM),
        scratch_shapes=[
            pltpu.VMEM((ROWS, COLS), jnp.bfloat16),
            *[
                pltpu.VMEM((rows_of(s), COLS), jnp.bfloat16)
                for s in range(LOG)
            ],
            *[
                pltpu.VMEM((rows_of(s), COLS), jnp.bfloat16)
                for s in range(LOG)
            ],
            pltpu.VMEM((ROWS, COLS), jnp.bfloat16),
            pltpu.SemaphoreType.DMA((4 * LOG,)),
            pltpu.SemaphoreType.DMA((4 * LOG,)),
        ],
    )(x, w_mat)


# device time: 53187 ns/iter; 3.3274x vs baseline; 1.1143x over previous
import jax
import jax.numpy as jnp
from jax import lax
from jax.experimental import pallas as pl
from jax.experimental.pallas import tpu as pltpu

N_DEV = 32
LOG = 5
ROWS = 1024
COLS = 1024
HALF_ROWS = ROWS // 2
CHUNK = HALF_ROWS // N_DEV


def kernel(x, w_mat):
    def body(
        x_ref, w_ref, out_ref,
        rsend,
        ra0, ra1, ra2, ra3, ra4,
        rb0, rb1, rb2, rb3, rb4,
        ag,
        send_sems, recv_sems,
    ):
        my = lax.axis_index("i")
        z = my >> 3
        idx = my & 7
        yc = idx >> 1
        xc = (idx & 1) ^ (yc & 1)

        recv_a = [ra0, ra1, ra2, ra3, ra4]
        recv_b = [rb0, rb1, rb2, rb3, rb4]

        py = yc ^ 1
        y1_partner = 8 * z + 2 * py + (xc ^ (py & 1))

        x_st = (my ^ 1, xc)
        z1_st = (my ^ 8, z & 1)
        y1_st = (y1_partner, yc & 1)
        z2_st = (my ^ 16, (z >> 1) & 1)
        y2_st = (my ^ 4, (yc >> 1) & 1)
        stages_a = [x_st, z1_st, y1_st, z2_st, y2_st]
        stages_b = [y1_st, x_st, z1_st, y2_st, z2_st]

        barrier = pltpu.get_barrier_semaphore()
        for partner, _ in stages_a:
            pl.semaphore_signal(
                barrier,
                inc=1,
                device_id=(partner,),
                device_id_type=pl.DeviceIdType.MESH,
            )
        pl.semaphore_wait(barrier, LOG)

        acc = jnp.dot(
            x_ref[...].astype(jnp.bfloat16),
            w_ref[...].astype(jnp.bfloat16),
            preferred_element_type=jnp.float32,
        )
        out_ref[...] = acc
        rsend[...] = acc.astype(jnp.bfloat16)

        def rs_start(s, off, stage, recv_ref, sem_base):
            partner, bit = stage
            half = HALF_ROWS >> (s + 1)
            send_off = pl.multiple_of(off + (1 - bit) * half, CHUNK)
            keep_off = pl.multiple_of(off + bit * half, CHUNK)
            rdma = pltpu.make_async_remote_copy(
                src_ref=rsend.at[pl.ds(send_off, half)],
                dst_ref=recv_ref,
                send_sem=send_sems.at[sem_base + s],
                recv_sem=recv_sems.at[sem_base + s],
                device_id=(partner,),
                device_id_type=pl.DeviceIdType.MESH,
            )
            rdma.start()
            return rdma, keep_off, half

        def rs_accum(s, keep_off, half, recv_ref):
            acc = (
                out_ref[pl.ds(keep_off, half), :]
                + recv_ref[...].astype(jnp.float32)
            )
            out_ref[pl.ds(keep_off, half), :] = acc
            stage_buf = ag if s == LOG - 1 else rsend
            stage_buf[pl.ds(keep_off, half), :] = acc.astype(jnp.bfloat16)

        off_a = jnp.int32(0)
        off_b = jnp.int32(HALF_ROWS)
        for s in range(LOG):
            rdma_a, keep_a, half = rs_start(s, off_a, stages_a[s], recv_a[s], 0)
            rdma_b, keep_b, _ = rs_start(s, off_b, stages_b[s], recv_b[s], LOG)
            rdma_a.wait()
            rs_accum(s, keep_a, half, recv_a[s])
            rdma_b.wait()
            rs_accum(s, keep_b, half, recv_b[s])
            off_a = keep_a
            off_b = keep_b

        def ag_start(t, off, stage, sem_base):
            partner, bit = stage
            ln = CHUNK << t
            rdma = pltpu.make_async_remote_copy(
                src_ref=ag.at[pl.ds(off, ln)],
                dst_ref=ag.at[pl.ds(off, ln)],
                send_sem=send_sems.at[2 * LOG + sem_base + t],
                recv_sem=recv_sems.at[2 * LOG + sem_base + t],
                device_id=(partner,),
                device_id_type=pl.DeviceIdType.MESH,
            )
            rdma.start()
            recv_off = pl.multiple_of(off + ln - 2 * bit * ln, CHUNK)
            return rdma, recv_off, pl.multiple_of(off - bit * ln, CHUNK)

        ag_a = list(reversed(stages_a))
        ag_b = list(reversed(stages_b))
        pending = None
        for t in range(LOG):
            ln = CHUNK << t
            rdma_a, rcv_a, new_off_a = ag_start(t, off_a, ag_a[t], 0)
            rdma_b, rcv_b, new_off_b = ag_start(t, off_b, ag_b[t], LOG)
            if pending is not None:
                po_a, po_b, pln = pending
                out_ref[pl.ds(po_a, pln), :] = ag[pl.ds(po_a, pln), :].astype(
                    jnp.float32
                )
                out_ref[pl.ds(po_b, pln), :] = ag[pl.ds(po_b, pln), :].astype(
                    jnp.float32
                )
            rdma_a.wait()
            rdma_b.wait()
            pending = (rcv_a, rcv_b, ln)
            off_a = new_off_a
            off_b = new_off_b
        po_a, po_b, pln = pending
        out_ref[pl.ds(po_a, pln), :] = ag[pl.ds(po_a, pln), :].astype(
            jnp.float32
        )
        out_ref[pl.ds(po_b, pln), :] = ag[pl.ds(po_b, pln), :].astype(
            jnp.float32
        )

    rows_of = lambda s: HALF_ROWS >> (s + 1)
    return pl.pallas_call(
        body,
        out_shape=jax.ShapeDtypeStruct((ROWS, COLS), jnp.float32),
        in_specs=[
            pl.BlockSpec(memory_space=pltpu.VMEM),
            pl.BlockSpec(memory_space=pltpu.VMEM),
        ],
        out_specs=pl.BlockSpec(memory_space=pltpu.VMEM),
        scratch_shapes=[
            pltpu.VMEM((ROWS, COLS), jnp.bfloat16),
            *[
                pltpu.VMEM((rows_of(s), COLS), jnp.bfloat16)
                for s in range(LOG)
            ],
            *[
                pltpu.VMEM((rows_of(s), COLS), jnp.bfloat16)
                for s in range(LOG)
            ],
            pltpu.VMEM((ROWS, COLS), jnp.bfloat16),
            pltpu.SemaphoreType.DMA((4 * LOG,)),
            pltpu.SemaphoreType.DMA((4 * LOG,)),
        ],
        compiler_params=pltpu.CompilerParams(collective_id=0),
    )(x, w_mat)


# device time: 52275 ns/iter; 3.3854x vs baseline; 1.0174x over previous
import jax
import jax.numpy as jnp
from jax import lax
from jax.experimental import pallas as pl
from jax.experimental.pallas import tpu as pltpu

N_DEV = 32
LOG = 5
ROWS = 1024
COLS = 1024
HALF_ROWS = ROWS // 2
CHUNK = HALF_ROWS // N_DEV


def kernel(x, w_mat):
    def body(
        x_ref, w_ref, out_ref,
        work,
        ra0, ra1, ra2, ra3, ra4,
        rb0, rb1, rb2, rb3, rb4,
        send_sems, recv_sems,
    ):
        my = lax.axis_index("i")
        z = my >> 3
        idx = my & 7
        yc = idx >> 1
        xc = (idx & 1) ^ (yc & 1)

        recv_a = [ra0, ra1, ra2, ra3, ra4]
        recv_b = [rb0, rb1, rb2, rb3, rb4]

        py = yc ^ 1
        y1_partner = 8 * z + 2 * py + (xc ^ (py & 1))

        x_st = (my ^ 1, xc)
        z1_st = (my ^ 8, z & 1)
        y1_st = (y1_partner, yc & 1)
        z2_st = (my ^ 16, (z >> 1) & 1)
        y2_st = (my ^ 4, (yc >> 1) & 1)
        stages_a = [x_st, z1_st, y1_st, z2_st, y2_st]
        stages_b = [y1_st, x_st, z1_st, y2_st, z2_st]

        barrier = pltpu.get_barrier_semaphore()
        for partner, _ in stages_a:
            pl.semaphore_signal(
                barrier,
                inc=1,
                device_id=(partner,),
                device_id_type=pl.DeviceIdType.MESH,
            )
        pl.semaphore_wait(barrier, LOG)

        work[...] = jnp.dot(
            x_ref[...].astype(jnp.bfloat16),
            w_ref[...].astype(jnp.bfloat16),
            preferred_element_type=jnp.float32,
        ).astype(jnp.bfloat16)

        def rs_start(s, off, stage, recv_ref, sem_base):
            partner, bit = stage
            half = HALF_ROWS >> (s + 1)
            send_off = pl.multiple_of(off + (1 - bit) * half, CHUNK)
            keep_off = pl.multiple_of(off + bit * half, CHUNK)
            rdma = pltpu.make_async_remote_copy(
                src_ref=work.at[pl.ds(send_off, half)],
                dst_ref=recv_ref,
                send_sem=send_sems.at[sem_base + s],
                recv_sem=recv_sems.at[sem_base + s],
                device_id=(partner,),
                device_id_type=pl.DeviceIdType.MESH,
            )
            rdma.start()
            return rdma, keep_off, half

        off_a = jnp.int32(0)
        off_b = jnp.int32(HALF_ROWS)
        rdma_a, keep_a, half = rs_start(0, off_a, stages_a[0], recv_a[0], 0)
        rdma_b, keep_b, _ = rs_start(0, off_b, stages_b[0], recv_b[0], LOG)
        for s in range(LOG):
            rdma_a.wait()
            work[pl.ds(keep_a, half), :] = (
                work[pl.ds(keep_a, half), :] + recv_a[s][...]
            )
            off_a = keep_a
            if s + 1 < LOG:
                rdma_a, keep_a, nhalf = rs_start(
                    s + 1, off_a, stages_a[s + 1], recv_a[s + 1], 0
                )
            rdma_b.wait()
            work[pl.ds(keep_b, half), :] = (
                work[pl.ds(keep_b, half), :] + recv_b[s][...]
            )
            off_b = keep_b
            if s + 1 < LOG:
                rdma_b, keep_b, _ = rs_start(
                    s + 1, off_b, stages_b[s + 1], recv_b[s + 1], LOG
                )
                half = nhalf

        def ag_start(t, off, stage, sem_base):
            partner, bit = stage
            ln = CHUNK << t
            rdma = pltpu.make_async_remote_copy(
                src_ref=work.at[pl.ds(off, ln)],
                dst_ref=work.at[pl.ds(off, ln)],
                send_sem=send_sems.at[2 * LOG + sem_base + t],
                recv_sem=recv_sems.at[2 * LOG + sem_base + t],
                device_id=(partner,),
                device_id_type=pl.DeviceIdType.MESH,
            )
            rdma.start()
            recv_off = pl.multiple_of(off + ln - 2 * bit * ln, CHUNK)
            return rdma, recv_off, pl.multiple_of(off - bit * ln, CHUNK)

        ag_a = list(reversed(stages_a))
        ag_b = list(reversed(stages_b))
        rdma_a, rcv_a, nxt_a = ag_start(0, off_a, ag_a[0], 0)
        rdma_b, rcv_b, nxt_b = ag_start(0, off_b, ag_b[0], LOG)
        out_ref[pl.ds(off_a, CHUNK), :] = work[pl.ds(off_a, CHUNK), :].astype(
            jnp.float32
        )
        out_ref[pl.ds(off_b, CHUNK), :] = work[pl.ds(off_b, CHUNK), :].astype(
            jnp.float32
        )
        for t in range(LOG):
            ln = CHUNK << t
            rdma_a.wait()
            off_a = nxt_a
            if t + 1 < LOG:
                rdma_a, na_rcv, nxt_a = ag_start(t + 1, off_a, ag_a[t + 1], 0)
            rdma_b.wait()
            off_b = nxt_b
            if t + 1 < LOG:
                rdma_b, nb_rcv, nxt_b = ag_start(t + 1, off_b, ag_b[t + 1], LOG)
            out_ref[pl.ds(rcv_a, ln), :] = work[pl.ds(rcv_a, ln), :].astype(
                jnp.float32
            )
            out_ref[pl.ds(rcv_b, ln), :] = work[pl.ds(rcv_b, ln), :].astype(
                jnp.float32
            )
            if t + 1 < LOG:
                rcv_a, rcv_b = na_rcv, nb_rcv

    rows_of = lambda s: HALF_ROWS >> (s + 1)
    return pl.pallas_call(
        body,
        out_shape=jax.ShapeDtypeStruct((ROWS, COLS), jnp.float32),
        in_specs=[
            pl.BlockSpec(memory_space=pltpu.VMEM),
            pl.BlockSpec(memory_space=pltpu.VMEM),
        ],
        out_specs=pl.BlockSpec(memory_space=pltpu.VMEM),
        scratch_shapes=[
            pltpu.VMEM((ROWS, COLS), jnp.bfloat16),
            *[
                pltpu.VMEM((rows_of(s), COLS), jnp.bfloat16)
                for s in range(LOG)
            ],
            *[
                pltpu.VMEM((rows_of(s), COLS), jnp.bfloat16)
                for s in range(LOG)
            ],
            pltpu.SemaphoreType.DMA((4 * LOG,)),
            pltpu.SemaphoreType.DMA((4 * LOG,)),
        ],
        compiler_params=pltpu.CompilerParams(collective_id=0),
    )(x, w_mat)


# device time: 48633 ns/iter; 3.6389x vs baseline; 1.0749x over previous
import jax
import jax.numpy as jnp
from jax import lax
from jax.experimental import pallas as pl
from jax.experimental.pallas import tpu as pltpu

N_DEV = 32
ROWS = 1024
COLS = 1024
HALF_ROWS = ROWS // 2
CHUNK = HALF_ROWS // N_DEV


def kernel(x, w_mat):
    def body(x_ref, w_ref, out_ref, work, *rest):
        (rax, raz1a, raz1b, raz1c, raz2a, raz2b, raz2c,
         rbz1a, rbz1b, rbz1c, rbz2a, rbz2b, rbz2c, rbx,
         send_sems, recv_sems) = rest

        my = lax.axis_index("i")
        z = my >> 3
        idx = my & 7
        yc = idx >> 1
        xc = (idx & 1) ^ (yc & 1)

        py1 = yc ^ 1
        y1_partner = 8 * z + 2 * py1 + (xc ^ (py1 & 1))
        px = my ^ 1
        zy1 = dict(
            bz=z & 1, by=yc & 1,
            pz=my ^ 8, py=y1_partner, pd=y1_partner ^ 8,
        )
        zy2 = dict(
            bz=(z >> 1) & 1, by=(yc >> 1) & 1,
            pz=my ^ 16, py=my ^ 4, pd=my ^ 20,
        )

        partners = [px, zy1["pz"], zy1["py"], zy1["pd"],
                    zy2["pz"], zy2["py"], zy2["pd"]]

        barrier = pltpu.get_barrier_semaphore()
        for partner in partners:
            pl.semaphore_signal(
                barrier,
                inc=1,
                device_id=(partner,),
                device_id_type=pl.DeviceIdType.MESH,
            )
        pl.semaphore_wait(barrier, len(partners))

        work[...] = jnp.dot(
            x_ref[...].astype(jnp.bfloat16),
            w_ref[...].astype(jnp.bfloat16),
            preferred_element_type=jnp.float32,
        ).astype(jnp.bfloat16)

        sem_ctr = [0]

        def rdma_to(src_off, n_rows, dst_ref, dst_off, partner):
            si = sem_ctr[0]
            sem_ctr[0] += 1
            src = work.at[pl.ds(pl.multiple_of(src_off, CHUNK), n_rows)]
            if dst_ref is None:
                dst = work.at[pl.ds(pl.multiple_of(dst_off, CHUNK), n_rows)]
            else:
                dst = dst_ref
            rdma = pltpu.make_async_remote_copy(
                src_ref=src,
                dst_ref=dst,
                send_sem=send_sems.at[si],
                recv_sem=recv_sems.at[si],
                device_id=(partner,),
                device_id_type=pl.DeviceIdType.MESH,
            )
            rdma.start()
            return rdma


        def rs_x_start(off, seg):
            half = seg // 2
            send_off = off + (1 - xc) * half
            keep_off = pl.multiple_of(off + xc * half, CHUNK)
            rbuf = rax if seg == HALF_ROWS else rbx
            rdma = rdma_to(send_off, half, rbuf, None, px)
            return [rdma], keep_off, half, [rbuf]

        def rs_zy_start(off, seg, g, bufs):
            qlen = seg // 4
            q = 2 * g["bz"] + g["by"]
            keep_off = pl.multiple_of(off + q * qlen, CHUNK)
            rdmas = [
                rdma_to(off + (q ^ 2) * qlen, qlen, bufs[0], None, g["pz"]),
                rdma_to(off + (q ^ 1) * qlen, qlen, bufs[1], None, g["py"]),
                rdma_to(off + (q ^ 3) * qlen, qlen, bufs[2], None, g["pd"]),
            ]
            return rdmas, keep_off, qlen, bufs

        def rs_finish(rdmas, keep_off, n_rows, bufs):
            for r in rdmas:
                r.wait()
            acc = work[pl.ds(keep_off, n_rows), :]
            for b in bufs:
                acc = acc + b[...]
            work[pl.ds(keep_off, n_rows), :] = acc
            return keep_off

        sa = rs_x_start(jnp.int32(0), HALF_ROWS)
        sb = rs_zy_start(jnp.int32(HALF_ROWS), HALF_ROWS, zy1,
                         [rbz1a, rbz1b, rbz1c])
        off_a = rs_finish(*sa)
        sa = rs_zy_start(off_a, HALF_ROWS // 2, zy1, [raz1a, raz1b, raz1c])
        off_b = rs_finish(*sb)
        sb = rs_zy_start(off_b, HALF_ROWS // 4, zy2, [rbz2a, rbz2b, rbz2c])
        off_a = rs_finish(*sa)
        sa = rs_zy_start(off_a, HALF_ROWS // 8, zy2, [raz2a, raz2b, raz2c])
        off_b = rs_finish(*sb)
        sb = rs_x_start(off_b, 2 * CHUNK)
        off_a = rs_finish(*sa)
        off_b = rs_finish(*sb)


        def ag_x_start(off, ln):
            base = pl.multiple_of(off - xc * ln, CHUNK)
            rdma = rdma_to(off, ln, None, off, px)
            recv = [(pl.multiple_of(base + (1 - xc) * ln, CHUNK), ln)]
            return [rdma], base, recv

        def ag_zy_start(off, ln, g):
            q = 2 * g["bz"] + g["by"]
            base = pl.multiple_of(off - q * ln, CHUNK)
            rdmas = [
                rdma_to(off, ln, None, off, g["pz"]),
                rdma_to(off, ln, None, off, g["py"]),
                rdma_to(off, ln, None, off, g["pd"]),
            ]
            recv = [
                (pl.multiple_of(base + (q ^ 2) * ln, CHUNK), ln),
                (pl.multiple_of(base + (q ^ 1) * ln, CHUNK), ln),
                (pl.multiple_of(base + (q ^ 3) * ln, CHUNK), ln),
            ]
            return rdmas, base, recv

        def ag_wait(rdmas):
            for r in rdmas:
                r.wait()

        def to_f32(ranges):
            for off, ln in ranges:
                out_ref[pl.ds(off, ln), :] = work[pl.ds(off, ln), :].astype(
                    jnp.float32
                )

        ga, base_a, pend_a = ag_zy_start(off_a, CHUNK, zy2)
        gb, base_b, pend_b = ag_x_start(off_b, CHUNK)
        to_f32([(off_a, CHUNK), (off_b, CHUNK)])
        ag_wait(ga)
        off_a = base_a
        ga, base_a, npend_a = ag_zy_start(off_a, 4 * CHUNK, zy1)
        ag_wait(gb)
        off_b = base_b
        gb, base_b, npend_b = ag_zy_start(off_b, 2 * CHUNK, zy2)
        to_f32(pend_a + pend_b)
        pend_a, pend_b = npend_a, npend_b
        ag_wait(ga)
        off_a = base_a
        ga, _, npend_a = ag_x_start(off_a, 16 * CHUNK)
        ag_wait(gb)
        off_b = base_b
        gb, _, npend_b = ag_zy_start(off_b, 8 * CHUNK, zy1)
        to_f32(pend_a + pend_b)
        ag_wait(ga)
        ag_wait(gb)
        to_f32(npend_a + npend_b)

    return pl.pallas_call(
        body,
        out_shape=jax.ShapeDtypeStruct((ROWS, COLS), jnp.float32),
        in_specs=[
            pl.BlockSpec(memory_space=pltpu.VMEM),
            pl.BlockSpec(memory_space=pltpu.VMEM),
        ],
        out_specs=pl.BlockSpec(memory_space=pltpu.VMEM),
        scratch_shapes=[
            pltpu.VMEM((ROWS, COLS), jnp.bfloat16),
            pltpu.VMEM((HALF_ROWS // 2, COLS), jnp.bfloat16),
            *[pltpu.VMEM((HALF_ROWS // 8, COLS), jnp.bfloat16)
              for _ in range(3)],
            *[pltpu.VMEM((HALF_ROWS // 32, COLS), jnp.bfloat16)
              for _ in range(3)],
            *[pltpu.VMEM((HALF_ROWS // 4, COLS), jnp.bfloat16)
              for _ in range(3)],
            *[pltpu.VMEM((HALF_ROWS // 16, COLS), jnp.bfloat16)
              for _ in range(3)],
            pltpu.VMEM((CHUNK, COLS), jnp.bfloat16),
            pltpu.SemaphoreType.DMA((28,)),
            pltpu.SemaphoreType.DMA((28,)),
        ],
        compiler_params=pltpu.CompilerParams(collective_id=0),
    )(x, w_mat)
